# baseline (speedup 1.0000x reference)
import functools

import jax
import jax.numpy as jnp
from jax.experimental import pallas as pl
from jax.experimental.pallas import tpu as pltpu

E = 8
EP = 128
TOP_K = 2
H = 2048
I = 4096
AUX_COEF = 0.01
T = 2048

BI = 128
NI = I // BI


def _router_kernel(x_ref, gw_ref, comb_ref, aux_ref):
    x = x_ref[...]
    gw = gw_ref[...]
    logits = jnp.dot(x, gw, preferred_element_type=jnp.float32)
    col = jax.lax.broadcasted_iota(jnp.int32, logits.shape, 1)
    valid = col < E
    logits = jnp.where(valid, logits, jnp.float32(-1e30))
    m = jnp.max(logits, axis=1, keepdims=True)
    ex = jnp.exp(logits - m)
    p = ex / jnp.sum(ex, axis=1, keepdims=True)
    m1 = jnp.max(p, axis=1, keepdims=True)
    e0 = jnp.min(jnp.where(p == m1, col, jnp.int32(2**30)), axis=1, keepdims=True)
    p_wo = jnp.where(col == e0, jnp.float32(-1.0), p)
    m2 = jnp.max(p_wo, axis=1, keepdims=True)
    e1 = jnp.min(jnp.where(p_wo == m2, col, jnp.int32(2**30)), axis=1, keepdims=True)
    s = m1 + m2
    w0 = m1 / s
    w1 = m2 / s
    oh0 = col == e0
    oh1 = col == e1
    comb = jnp.where(oh0, w0, 0.0) + jnp.where(oh1, w1, 0.0)
    comb_ref[...] = comb
    counts = jnp.sum(jnp.where(oh0, 1.0, 0.0) + jnp.where(oh1, 1.0, 0.0),
                     axis=0, keepdims=True)
    f = counts / jnp.float32(T)
    pmean = jnp.mean(p, axis=0, keepdims=True)
    aux = jnp.sum(f * pmean, axis=1, keepdims=True) * jnp.float32(E * AUX_COEF)
    aux_ref[...] = aux


def _ffn_dense_kernel(comb_ref, x_ref, wg_ref, wu_ref, wd_ref, out_ref):
    e = pl.program_id(0)
    i = pl.program_id(1)
    x = x_ref[...]
    g = jnp.dot(x, wg_ref[0], preferred_element_type=jnp.float32)
    u = jnp.dot(x, wu_ref[0], preferred_element_type=jnp.float32)
    h = g * jax.nn.sigmoid(g) * u
    y = jnp.dot(h, wd_ref[0], preferred_element_type=jnp.float32)
    comb = comb_ref[...]
    col = jax.lax.broadcasted_iota(jnp.int32, comb.shape, 1)
    ce = jnp.sum(jnp.where(col == e, comb, 0.0), axis=1, keepdims=True)
    contrib = y * ce

    @pl.when((e == 0) & (i == 0))
    def _init():
        out_ref[...] = contrib

    @pl.when((e > 0) | (i > 0))
    def _acc():
        out_ref[...] += contrib


def kernel(hidden_states, gate_w, w_gate, w_up, w_down):
    B, S, Hd = hidden_states.shape
    x = hidden_states.reshape(-1, Hd)
    gw_pad = jnp.zeros((Hd, EP), dtype=gate_w.dtype).at[:, :E].set(gate_w)

    comb, aux = pl.pallas_call(
        _router_kernel,
        out_shape=[
            jax.ShapeDtypeStruct((T, EP), jnp.float32),
            jax.ShapeDtypeStruct((1, 1), jnp.float32),
        ],
        in_specs=[
            pl.BlockSpec((T, Hd), lambda: (0, 0)),
            pl.BlockSpec((Hd, EP), lambda: (0, 0)),
        ],
        out_specs=[
            pl.BlockSpec((T, EP), lambda: (0, 0)),
            pl.BlockSpec((1, 1), lambda: (0, 0)),
        ],
    )(x, gw_pad)

    out = pl.pallas_call(
        _ffn_dense_kernel,
        grid=(E, NI),
        out_shape=jax.ShapeDtypeStruct((T, Hd), jnp.float32),
        in_specs=[
            pl.BlockSpec((T, EP), lambda e, i: (0, 0)),
            pl.BlockSpec((T, Hd), lambda e, i: (0, 0)),
            pl.BlockSpec((1, Hd, BI), lambda e, i: (e, 0, i)),
            pl.BlockSpec((1, Hd, BI), lambda e, i: (e, 0, i)),
            pl.BlockSpec((1, BI, Hd), lambda e, i: (e, i, 0)),
        ],
        out_specs=pl.BlockSpec((T, Hd), lambda e, i: (0, 0)),
    )(comb, x, w_gate, w_up, w_down)

    return out.reshape(B, S, Hd), aux.reshape(())


# speedup vs baseline: 1.5559x; 1.5559x over previous
import functools

import jax
import jax.numpy as jnp
from jax.experimental import pallas as pl
from jax.experimental.pallas import tpu as pltpu

E = 8
EP = 128
TOP_K = 2
H = 2048
I = 4096
AUX_COEF = 0.01
T = 2048

BI = 512
NI = I // BI


def _router_kernel(x_ref, gw_ref, comb_ref, aux_ref):
    x = x_ref[...]
    gw = gw_ref[...]
    logits = jnp.dot(x, gw, preferred_element_type=jnp.float32)
    col = jax.lax.broadcasted_iota(jnp.int32, logits.shape, 1)
    valid = col < E
    logits = jnp.where(valid, logits, jnp.float32(-1e30))
    m = jnp.max(logits, axis=1, keepdims=True)
    ex = jnp.exp(logits - m)
    p = ex / jnp.sum(ex, axis=1, keepdims=True)
    m1 = jnp.max(p, axis=1, keepdims=True)
    e0 = jnp.min(jnp.where(p == m1, col, jnp.int32(2**30)), axis=1, keepdims=True)
    p_wo = jnp.where(col == e0, jnp.float32(-1.0), p)
    m2 = jnp.max(p_wo, axis=1, keepdims=True)
    e1 = jnp.min(jnp.where(p_wo == m2, col, jnp.int32(2**30)), axis=1, keepdims=True)
    s = m1 + m2
    w0 = m1 / s
    w1 = m2 / s
    oh0 = col == e0
    oh1 = col == e1
    comb = jnp.where(oh0, w0, 0.0) + jnp.where(oh1, w1, 0.0)
    comb_ref[...] = comb
    counts = jnp.sum(jnp.where(oh0, 1.0, 0.0) + jnp.where(oh1, 1.0, 0.0),
                     axis=0, keepdims=True)
    f = counts / jnp.float32(T)
    pmean = jnp.mean(p, axis=0, keepdims=True)
    aux = jnp.sum(f * pmean, axis=1, keepdims=True) * jnp.float32(E * AUX_COEF)
    aux_ref[...] = aux


def _ffn_dense_kernel(comb_ref, x_ref, wg_ref, wu_ref, wd_ref, out_ref):
    e = pl.program_id(0)
    i = pl.program_id(1)
    x = x_ref[...]
    g = jnp.dot(x, wg_ref[0], preferred_element_type=jnp.float32)
    u = jnp.dot(x, wu_ref[0], preferred_element_type=jnp.float32)
    h = g * jax.nn.sigmoid(g) * u
    y = jnp.dot(h.astype(jnp.bfloat16), wd_ref[0],
                preferred_element_type=jnp.float32)
    comb = comb_ref[...]
    col = jax.lax.broadcasted_iota(jnp.int32, comb.shape, 1)
    ce = jnp.sum(jnp.where(col == e, comb, 0.0), axis=1, keepdims=True)
    contrib = y * ce

    @pl.when((e == 0) & (i == 0))
    def _init():
        out_ref[...] = contrib

    @pl.when((e > 0) | (i > 0))
    def _acc():
        out_ref[...] += contrib


def kernel(hidden_states, gate_w, w_gate, w_up, w_down):
    B, S, Hd = hidden_states.shape
    x = hidden_states.reshape(-1, Hd)
    gw_pad = jnp.zeros((Hd, EP), dtype=gate_w.dtype).at[:, :E].set(gate_w)

    comb, aux = pl.pallas_call(
        _router_kernel,
        out_shape=[
            jax.ShapeDtypeStruct((T, EP), jnp.float32),
            jax.ShapeDtypeStruct((1, 1), jnp.float32),
        ],
        in_specs=[
            pl.BlockSpec((T, Hd), lambda: (0, 0)),
            pl.BlockSpec((Hd, EP), lambda: (0, 0)),
        ],
        out_specs=[
            pl.BlockSpec((T, EP), lambda: (0, 0)),
            pl.BlockSpec((1, 1), lambda: (0, 0)),
        ],
    )(x, gw_pad)

    x_b = x.astype(jnp.bfloat16)
    wg_b = w_gate.astype(jnp.bfloat16)
    wu_b = w_up.astype(jnp.bfloat16)
    wd_b = w_down.astype(jnp.bfloat16)
    out = pl.pallas_call(
        _ffn_dense_kernel,
        grid=(E, NI),
        out_shape=jax.ShapeDtypeStruct((T, Hd), jnp.float32),
        in_specs=[
            pl.BlockSpec((T, EP), lambda e, i: (0, 0)),
            pl.BlockSpec((T, Hd), lambda e, i: (0, 0)),
            pl.BlockSpec((1, Hd, BI), lambda e, i: (e, 0, i)),
            pl.BlockSpec((1, Hd, BI), lambda e, i: (e, 0, i)),
            pl.BlockSpec((1, BI, Hd), lambda e, i: (e, i, 0)),
        ],
        out_specs=pl.BlockSpec((T, Hd), lambda e, i: (0, 0)),
    )(comb, x_b, wg_b, wu_b, wd_b)

    return out.reshape(B, S, Hd), aux.reshape(())


# speedup vs baseline: 2.4560x; 1.5785x over previous
import jax
import jax.numpy as jnp
from jax.experimental import pallas as pl
from jax.experimental.pallas import tpu as pltpu

E = 8
EP = 128
H = 2048
I = 4096
AUX_COEF = 0.01
T = 2048

BT = 512
NB = T * 2 // BT + E
NP = NB * BT
BI = 256
NI = I // BI


def _router_kernel(x_ref, gw_ref, posw_ref, counts_ref, aux_ref):
    x = x_ref[...]
    gw = gw_ref[...]
    logits = jnp.dot(x, gw, preferred_element_type=jnp.float32)
    col = jax.lax.broadcasted_iota(jnp.int32, logits.shape, 1)
    valid = col < E
    logits = jnp.where(valid, logits, jnp.float32(-1e30))
    m = jnp.max(logits, axis=1, keepdims=True)
    ex = jnp.exp(logits - m)
    p = ex / jnp.sum(ex, axis=1, keepdims=True)
    m1 = jnp.max(p, axis=1, keepdims=True)
    e0 = jnp.min(jnp.where(p == m1, col, jnp.int32(2**30)), axis=1, keepdims=True)
    p_wo = jnp.where(col == e0, jnp.float32(-1.0), p)
    m2 = jnp.max(p_wo, axis=1, keepdims=True)
    e1 = jnp.min(jnp.where(p_wo == m2, col, jnp.int32(2**30)), axis=1, keepdims=True)
    s = m1 + m2
    w0 = m1 / s
    w1 = m2 / s
    oh0 = col == e0
    oh1 = col == e1
    ohf = jnp.where(oh0, 1.0, 0.0) + jnp.where(oh1, 1.0, 0.0)
    counts = jnp.sum(ohf, axis=0, keepdims=True)
    counts_ref[...] = counts
    f = counts / jnp.float32(T)
    pmean = jnp.mean(p, axis=0, keepdims=True)
    aux_ref[...] = jnp.sum(f * pmean, axis=1, keepdims=True) * jnp.float32(E * AUX_COEF)
    r_io = jax.lax.broadcasted_iota(jnp.int32, (T, T), 0)
    c_io = jax.lax.broadcasted_iota(jnp.int32, (T, T), 1)
    ltri = jnp.where(c_io < r_io, 1.0, 0.0)
    cnt_str = jnp.dot(ltri, ohf, preferred_element_type=jnp.float32)
    r0 = jnp.sum(jnp.where(oh0, cnt_str, 0.0), axis=1, keepdims=True)
    r1 = jnp.sum(jnp.where(oh1, cnt_str, 0.0), axis=1, keepdims=True)
    pad_cnt = jnp.ceil(counts / jnp.float32(BT)) * jnp.float32(BT)
    a_io = jax.lax.broadcasted_iota(jnp.int32, (EP, EP), 0)
    b_io = jax.lax.broadcasted_iota(jnp.int32, (EP, EP), 1)
    utri = jnp.where(a_io < b_io, 1.0, 0.0)
    offs = jnp.dot(pad_cnt, utri, preferred_element_type=jnp.float32)
    off0 = jnp.sum(jnp.where(oh0, offs, 0.0), axis=1, keepdims=True)
    off1 = jnp.sum(jnp.where(oh1, offs, 0.0), axis=1, keepdims=True)
    pos0 = off0 + r0
    pos1 = off1 + r1
    colw = jax.lax.broadcasted_iota(jnp.int32, (T, EP), 1)
    posw_ref[...] = (jnp.where(colw == 0, pos0, 0.0)
                     + jnp.where(colw == 1, pos1, 0.0)
                     + jnp.where(colw == 2, w0, 0.0)
                     + jnp.where(colw == 3, w1, 0.0))


def _ffn_kernel(be_ref, posw_ref, x_ref, wg_ref, wu_ref, wd_ref, out_ref,
                p_ref, xs_ref, ws_ref, y_ref):
    b = pl.program_id(0)
    i = pl.program_id(1)

    @pl.when((b == 0) & (i == 0))
    def _zero_out():
        out_ref[...] = jnp.zeros_like(out_ref)

    @pl.when(i == 0)
    def _gather():
        posw = posw_ref[...]
        pos0 = posw[:, 0:1]
        pos1 = posw[:, 1:2]
        w0 = posw[:, 2:3]
        w1 = posw[:, 3:4]
        jj = (jax.lax.broadcasted_iota(jnp.int32, (T, BT), 1)
              + b * BT).astype(jnp.float32)
        m0 = pos0 == jj
        m1 = pos1 == jj
        pmat = jnp.where(m0, 1.0, 0.0) + jnp.where(m1, 1.0, 0.0)
        p_ref[...] = pmat.astype(jnp.bfloat16)
        a = jnp.where(m0, w0, 0.0) + jnp.where(m1, w1, 0.0)
        ones = jnp.ones((T, 1), jnp.float32)
        ws_ref[...] = jax.lax.dot_general(
            a, ones, (((0,), (0,)), ((), ())),
            preferred_element_type=jnp.float32)
        xs = jax.lax.dot_general(
            pmat.astype(jnp.bfloat16), x_ref[...], (((0,), (0,)), ((), ())),
            preferred_element_type=jnp.float32)
        xs_ref[...] = xs.astype(jnp.bfloat16)

    xs = xs_ref[...]
    wg = wg_ref[0].astype(jnp.bfloat16)
    wu = wu_ref[0].astype(jnp.bfloat16)
    g = jnp.dot(xs, wg, preferred_element_type=jnp.float32)
    u = jnp.dot(xs, wu, preferred_element_type=jnp.float32)
    h = g * jax.nn.sigmoid(g) * u
    yi = jnp.dot(h.astype(jnp.bfloat16), wd_ref[0].astype(jnp.bfloat16),
                 preferred_element_type=jnp.float32)

    @pl.when(i == 0)
    def _y_init():
        y_ref[...] = yi

    @pl.when(i > 0)
    def _y_acc():
        y_ref[...] += yi

    @pl.when(i == NI - 1)
    def _scatter():
        yw = (y_ref[...] * ws_ref[...]).astype(jnp.bfloat16)
        out_ref[...] += jnp.dot(p_ref[...], yw,
                                preferred_element_type=jnp.float32)


def kernel(hidden_states, gate_w, w_gate, w_up, w_down):
    B, S, Hd = hidden_states.shape
    x = hidden_states.reshape(-1, Hd)
    gw_pad = jnp.zeros((Hd, EP), dtype=gate_w.dtype).at[:, :E].set(gate_w)

    posw, counts, aux = pl.pallas_call(
        _router_kernel,
        out_shape=[
            jax.ShapeDtypeStruct((T, EP), jnp.float32),
            jax.ShapeDtypeStruct((1, EP), jnp.float32),
            jax.ShapeDtypeStruct((1, 1), jnp.float32),
        ],
        in_specs=[
            pl.BlockSpec((T, Hd), lambda: (0, 0)),
            pl.BlockSpec((Hd, EP), lambda: (0, 0)),
        ],
        out_specs=[
            pl.BlockSpec((T, EP), lambda: (0, 0)),
            pl.BlockSpec((1, EP), lambda: (0, 0)),
            pl.BlockSpec((1, 1), lambda: (0, 0)),
        ],
    )(x, gw_pad)

    pad_cnt = (jnp.ceil(counts[0, :E] / BT) * BT).astype(jnp.int32)
    ends = jnp.cumsum(pad_cnt)
    starts = jnp.arange(NB, dtype=jnp.int32) * BT
    be = jnp.sum((starts[:, None] >= ends[None, :]).astype(jnp.int32), axis=1)
    be = jnp.minimum(be, E - 1).astype(jnp.int32)

    x_b = x.astype(jnp.bfloat16)
    grid_spec = pltpu.PrefetchScalarGridSpec(
        num_scalar_prefetch=1,
        grid=(NB, NI),
        in_specs=[
            pl.BlockSpec((T, EP), lambda b, i, be: (0, 0)),
            pl.BlockSpec((T, Hd), lambda b, i, be: (0, 0)),
            pl.BlockSpec((1, Hd, BI), lambda b, i, be: (be[b], 0, i)),
            pl.BlockSpec((1, Hd, BI), lambda b, i, be: (be[b], 0, i)),
            pl.BlockSpec((1, BI, Hd), lambda b, i, be: (be[b], i, 0)),
        ],
        out_specs=pl.BlockSpec((T, Hd), lambda b, i, be: (0, 0)),
        scratch_shapes=[
            pltpu.VMEM((T, BT), jnp.bfloat16),
            pltpu.VMEM((BT, Hd), jnp.bfloat16),
            pltpu.VMEM((BT, 1), jnp.float32),
            pltpu.VMEM((BT, Hd), jnp.float32),
        ],
    )
    out = pl.pallas_call(
        _ffn_kernel,
        grid_spec=grid_spec,
        out_shape=jax.ShapeDtypeStruct((T, Hd), jnp.float32),
    )(be, posw, x_b, w_gate, w_up, w_down)

    return out.reshape(B, S, Hd), aux.reshape(())


# speedup vs baseline: 2.4897x; 1.0137x over previous
import jax
import jax.numpy as jnp
from jax.experimental import pallas as pl
from jax.experimental.pallas import tpu as pltpu

E = 8
EP = 128
H = 2048
I = 4096
AUX_COEF = 0.01
T = 2048

BT = 1024
NB = T * 2 // BT + E
NP = NB * BT
BI = 256
NI = I // BI
BC = 256


def _router_kernel(x_ref, gw_ref, posw_ref, counts_ref, aux_ref):
    x = x_ref[...]
    gw = gw_ref[...]
    logits = jnp.dot(x, gw, preferred_element_type=jnp.float32)
    col = jax.lax.broadcasted_iota(jnp.int32, logits.shape, 1)
    valid = col < E
    logits = jnp.where(valid, logits, jnp.float32(-1e30))
    m = jnp.max(logits, axis=1, keepdims=True)
    ex = jnp.exp(logits - m)
    p = ex / jnp.sum(ex, axis=1, keepdims=True)
    m1 = jnp.max(p, axis=1, keepdims=True)
    e0 = jnp.min(jnp.where(p == m1, col, jnp.int32(2**30)), axis=1, keepdims=True)
    p_wo = jnp.where(col == e0, jnp.float32(-1.0), p)
    m2 = jnp.max(p_wo, axis=1, keepdims=True)
    e1 = jnp.min(jnp.where(p_wo == m2, col, jnp.int32(2**30)), axis=1, keepdims=True)
    s = m1 + m2
    w0 = m1 / s
    w1 = m2 / s
    oh0 = col == e0
    oh1 = col == e1
    ohf = jnp.where(oh0, 1.0, 0.0) + jnp.where(oh1, 1.0, 0.0)
    counts = jnp.sum(ohf, axis=0, keepdims=True)
    counts_ref[...] = counts
    f = counts / jnp.float32(T)
    pmean = jnp.mean(p, axis=0, keepdims=True)
    aux_ref[...] = jnp.sum(f * pmean, axis=1, keepdims=True) * jnp.float32(E * AUX_COEF)
    r_io = jax.lax.broadcasted_iota(jnp.int32, (T, T), 0)
    c_io = jax.lax.broadcasted_iota(jnp.int32, (T, T), 1)
    ltri = jnp.where(c_io < r_io, 1.0, 0.0)
    cnt_str = jnp.dot(ltri, ohf, preferred_element_type=jnp.float32)
    r0 = jnp.sum(jnp.where(oh0, cnt_str, 0.0), axis=1, keepdims=True)
    r1 = jnp.sum(jnp.where(oh1, cnt_str, 0.0), axis=1, keepdims=True)
    pad_cnt = jnp.ceil(counts / jnp.float32(BT)) * jnp.float32(BT)
    a_io = jax.lax.broadcasted_iota(jnp.int32, (EP, EP), 0)
    b_io = jax.lax.broadcasted_iota(jnp.int32, (EP, EP), 1)
    utri = jnp.where(a_io < b_io, 1.0, 0.0)
    offs = jnp.dot(pad_cnt, utri, preferred_element_type=jnp.float32)
    off0 = jnp.sum(jnp.where(oh0, offs, 0.0), axis=1, keepdims=True)
    off1 = jnp.sum(jnp.where(oh1, offs, 0.0), axis=1, keepdims=True)
    pos0 = off0 + r0
    pos1 = off1 + r1
    colw = jax.lax.broadcasted_iota(jnp.int32, (T, EP), 1)
    posw_ref[...] = (jnp.where(colw == 0, pos0, 0.0)
                     + jnp.where(colw == 1, pos1, 0.0)
                     + jnp.where(colw == 2, w0, 0.0)
                     + jnp.where(colw == 3, w1, 0.0))


def _ffn_kernel(sp_ref, posw_ref, x_ref, wg_ref, wu_ref, wd_ref, yw_ref,
                xs_ref, ws_ref, y_ref):
    b = pl.program_id(0)
    i = pl.program_id(1)
    live = sp_ref[1, b] == 1

    @pl.when(~live & (i == 0))
    def _dead():
        for zc in range(4):
            yw_ref[:, zc * (H // 4):(zc + 1) * (H // 4)] = jnp.zeros(
                (BT, H // 4), jnp.bfloat16)

    @pl.when(live & (i == 0))
    def _gather():
        posw = posw_ref[...]
        pos0 = posw[:, 0:1]
        pos1 = posw[:, 1:2]
        w0 = posw[:, 2:3]
        w1 = posw[:, 3:4]
        ones = jnp.ones((T, 1), jnp.float32)
        for c in range(BT // BC):
            jj = (jax.lax.broadcasted_iota(jnp.int32, (T, BC), 1)
                  + (b * BT + c * BC)).astype(jnp.float32)
            m0 = pos0 == jj
            m1 = pos1 == jj
            pmat = (jnp.where(m0, 1.0, 0.0)
                    + jnp.where(m1, 1.0, 0.0)).astype(jnp.bfloat16)
            a = jnp.where(m0, w0, 0.0) + jnp.where(m1, w1, 0.0)
            ws_ref[c * BC:(c + 1) * BC, :] = jax.lax.dot_general(
                a, ones, (((0,), (0,)), ((), ())),
                preferred_element_type=jnp.float32)
            xs = jax.lax.dot_general(
                pmat, x_ref[...], (((0,), (0,)), ((), ())),
                preferred_element_type=jnp.float32)
            xs_ref[c * BC:(c + 1) * BC, :] = xs.astype(jnp.bfloat16)

    @pl.when(live)
    def _ffn():
        xs = xs_ref[...]
        wg = wg_ref[0].astype(jnp.bfloat16)
        wu = wu_ref[0].astype(jnp.bfloat16)
        g = jnp.dot(xs, wg, preferred_element_type=jnp.float32)
        u = jnp.dot(xs, wu, preferred_element_type=jnp.float32)
        h = g * jax.nn.sigmoid(g) * u
        yi = jnp.dot(h.astype(jnp.bfloat16), wd_ref[0].astype(jnp.bfloat16),
                     preferred_element_type=jnp.float32)

        @pl.when(i == 0)
        def _y_init():
            y_ref[...] = yi

        @pl.when(i > 0)
        def _y_acc():
            y_ref[...] += yi

        @pl.when(i == NI - 1)
        def _weight_rows():
            ws = ws_ref[...]
            for hc in range(2):
                sl = slice(hc * (H // 2), (hc + 1) * (H // 2))
                yw_ref[:, sl] = (y_ref[:, sl] * ws).astype(jnp.bfloat16)


def _scatter_kernel(sp_ref, posw_ref, yw_ref, out_ref):
    b = pl.program_id(0)
    live = sp_ref[1, b] == 1

    @pl.when(b == 0)
    def _zero_out():
        for zc in range(8):
            out_ref[:, zc * (H // 8):(zc + 1) * (H // 8)] = jnp.zeros(
                (T, H // 8), jnp.float32)

    @pl.when(live)
    def _scatter():
        posw = posw_ref[...]
        pos0 = posw[:, 0:1]
        pos1 = posw[:, 1:2]
        for c in range(BT // BC):
            jj = (jax.lax.broadcasted_iota(jnp.int32, (T, BC), 1)
                  + (b * BT + c * BC)).astype(jnp.float32)
            pmat = (jnp.where(pos0 == jj, 1.0, 0.0)
                    + jnp.where(pos1 == jj, 1.0, 0.0)).astype(jnp.bfloat16)
            for hc in range(2):
                sl = slice(hc * (H // 2), (hc + 1) * (H // 2))
                out_ref[:, sl] += jnp.dot(
                    pmat, yw_ref[c * BC:(c + 1) * BC, sl],
                    preferred_element_type=jnp.float32)


def kernel(hidden_states, gate_w, w_gate, w_up, w_down):
    B, S, Hd = hidden_states.shape
    x = hidden_states.reshape(-1, Hd)
    gw_pad = jnp.zeros((Hd, EP), dtype=gate_w.dtype).at[:, :E].set(gate_w)

    posw, counts, aux = pl.pallas_call(
        _router_kernel,
        out_shape=[
            jax.ShapeDtypeStruct((T, EP), jnp.float32),
            jax.ShapeDtypeStruct((1, EP), jnp.float32),
            jax.ShapeDtypeStruct((1, 1), jnp.float32),
        ],
        in_specs=[
            pl.BlockSpec((T, Hd), lambda: (0, 0)),
            pl.BlockSpec((Hd, EP), lambda: (0, 0)),
        ],
        out_specs=[
            pl.BlockSpec((T, EP), lambda: (0, 0)),
            pl.BlockSpec((1, EP), lambda: (0, 0)),
            pl.BlockSpec((1, 1), lambda: (0, 0)),
        ],
    )(x, gw_pad)

    pad_cnt = (jnp.ceil(counts[0, :E] / BT) * BT).astype(jnp.int32)
    ends = jnp.cumsum(pad_cnt)
    starts = jnp.arange(NB, dtype=jnp.int32) * BT
    be = jnp.sum((starts[:, None] >= ends[None, :]).astype(jnp.int32), axis=1)
    be = jnp.minimum(be, E - 1).astype(jnp.int32)
    live = (starts < ends[E - 1]).astype(jnp.int32)
    sp = jnp.stack([be, live])

    x_b = x.astype(jnp.bfloat16)
    grid_spec = pltpu.PrefetchScalarGridSpec(
        num_scalar_prefetch=1,
        grid=(NB, NI),
        in_specs=[
            pl.BlockSpec((T, EP), lambda b, i, sp: (0, 0)),
            pl.BlockSpec((T, Hd), lambda b, i, sp: (0, 0)),
            pl.BlockSpec((1, Hd, BI), lambda b, i, sp: (sp[0, b], 0, i * sp[1, b])),
            pl.BlockSpec((1, Hd, BI), lambda b, i, sp: (sp[0, b], 0, i * sp[1, b])),
            pl.BlockSpec((1, BI, Hd), lambda b, i, sp: (sp[0, b], i * sp[1, b], 0)),
        ],
        out_specs=pl.BlockSpec((BT, Hd), lambda b, i, sp: (b, 0)),
        scratch_shapes=[
            pltpu.VMEM((BT, Hd), jnp.bfloat16),
            pltpu.VMEM((BT, 1), jnp.float32),
            pltpu.VMEM((BT, Hd), jnp.float32),
        ],
    )
    yw = pl.pallas_call(
        _ffn_kernel,
        grid_spec=grid_spec,
        out_shape=jax.ShapeDtypeStruct((NP, Hd), jnp.bfloat16),
    )(sp, posw, x_b, w_gate, w_up, w_down)

    scatter_spec = pltpu.PrefetchScalarGridSpec(
        num_scalar_prefetch=1,
        grid=(NB,),
        in_specs=[
            pl.BlockSpec((T, EP), lambda b, sp: (0, 0)),
            pl.BlockSpec((BT, Hd), lambda b, sp: (b, 0)),
        ],
        out_specs=pl.BlockSpec((T, Hd), lambda b, sp: (0, 0)),
    )
    out = pl.pallas_call(
        _scatter_kernel,
        grid_spec=scatter_spec,
        out_shape=jax.ShapeDtypeStruct((T, Hd), jnp.float32),
    )(sp, posw, yw)

    return out.reshape(B, S, Hd), aux.reshape(())


# speedup vs baseline: 2.6399x; 1.0603x over previous
import jax
import jax.numpy as jnp
from jax.experimental import pallas as pl
from jax.experimental.pallas import tpu as pltpu

E = 8
EP = 128
H = 2048
I = 4096
AUX_COEF = 0.01
T = 2048

BT = 1024
NB = T * 2 // BT + E
NP = NB * BT
BI = 512
NI = I // BI
BC = 256


def _router_kernel(x_ref, gw_ref, posw_ref, counts_ref, aux_ref):
    x = x_ref[...]
    gw = gw_ref[...]
    logits = jnp.dot(x, gw, preferred_element_type=jnp.float32)
    col = jax.lax.broadcasted_iota(jnp.int32, logits.shape, 1)
    valid = col < E
    logits = jnp.where(valid, logits, jnp.float32(-1e30))
    m = jnp.max(logits, axis=1, keepdims=True)
    ex = jnp.exp(logits - m)
    p = ex / jnp.sum(ex, axis=1, keepdims=True)
    m1 = jnp.max(p, axis=1, keepdims=True)
    e0 = jnp.min(jnp.where(p == m1, col, jnp.int32(2**30)), axis=1, keepdims=True)
    p_wo = jnp.where(col == e0, jnp.float32(-1.0), p)
    m2 = jnp.max(p_wo, axis=1, keepdims=True)
    e1 = jnp.min(jnp.where(p_wo == m2, col, jnp.int32(2**30)), axis=1, keepdims=True)
    s = m1 + m2
    w0 = m1 / s
    w1 = m2 / s
    oh0 = col == e0
    oh1 = col == e1
    ohf = jnp.where(oh0, 1.0, 0.0) + jnp.where(oh1, 1.0, 0.0)
    counts = jnp.sum(ohf, axis=0, keepdims=True)
    counts_ref[...] = counts
    f = counts / jnp.float32(T)
    pmean = jnp.mean(p, axis=0, keepdims=True)
    aux_ref[...] = jnp.sum(f * pmean, axis=1, keepdims=True) * jnp.float32(E * AUX_COEF)
    r_io = jax.lax.broadcasted_iota(jnp.int32, (T, T), 0)
    c_io = jax.lax.broadcasted_iota(jnp.int32, (T, T), 1)
    ltri = jnp.where(c_io < r_io, 1.0, 0.0)
    cnt_str = jnp.dot(ltri, ohf, preferred_element_type=jnp.float32)
    r0 = jnp.sum(jnp.where(oh0, cnt_str, 0.0), axis=1, keepdims=True)
    r1 = jnp.sum(jnp.where(oh1, cnt_str, 0.0), axis=1, keepdims=True)
    pad_cnt = jnp.ceil(counts / jnp.float32(BT)) * jnp.float32(BT)
    a_io = jax.lax.broadcasted_iota(jnp.int32, (EP, EP), 0)
    b_io = jax.lax.broadcasted_iota(jnp.int32, (EP, EP), 1)
    utri = jnp.where(a_io < b_io, 1.0, 0.0)
    offs = jnp.dot(pad_cnt, utri, preferred_element_type=jnp.float32)
    off0 = jnp.sum(jnp.where(oh0, offs, 0.0), axis=1, keepdims=True)
    off1 = jnp.sum(jnp.where(oh1, offs, 0.0), axis=1, keepdims=True)
    pos0 = off0 + r0
    pos1 = off1 + r1
    colw = jax.lax.broadcasted_iota(jnp.int32, (T, EP), 1)
    posw_ref[...] = (jnp.where(colw == 0, pos0, 0.0)
                     + jnp.where(colw == 1, pos1, 0.0)
                     + jnp.where(colw == 2, w0, 0.0)
                     + jnp.where(colw == 3, w1, 0.0))


def _gather_kernel(sp_ref, posw_ref, x_ref, xs_ref, ws_ref):
    b = pl.program_id(0)
    live = sp_ref[1, b] == 1

    @pl.when(live)
    def _gather():
        posw = posw_ref[...]
        pos0 = posw[:, 0:1]
        pos1 = posw[:, 1:2]
        w0 = posw[:, 2:3]
        w1 = posw[:, 3:4]
        ones = jnp.ones((T, 1), jnp.float32)
        for c in range(BT // BC):
            jj = (jax.lax.broadcasted_iota(jnp.int32, (T, BC), 1)
                  + (b * BT + c * BC)).astype(jnp.float32)
            m0 = pos0 == jj
            m1 = pos1 == jj
            pmat = (jnp.where(m0, 1.0, 0.0)
                    + jnp.where(m1, 1.0, 0.0)).astype(jnp.bfloat16)
            a = jnp.where(m0, w0, 0.0) + jnp.where(m1, w1, 0.0)
            ws_ref[c * BC:(c + 1) * BC, :] = jax.lax.dot_general(
                a, ones, (((0,), (0,)), ((), ())),
                preferred_element_type=jnp.float32)
            xs = jax.lax.dot_general(
                pmat, x_ref[...], (((0,), (0,)), ((), ())),
                preferred_element_type=jnp.float32)
            xs_ref[c * BC:(c + 1) * BC, :] = xs.astype(jnp.bfloat16)


def _ffn_kernel(sp_ref, xs_in_ref, ws_in_ref, wg_ref, wu_ref, wd_ref, yw_ref,
                y_ref):
    b = pl.program_id(0)
    i = pl.program_id(1)
    live = sp_ref[1, b] == 1

    @pl.when(~live & (i == 0))
    def _dead():
        for zc in range(4):
            yw_ref[:, zc * (H // 4):(zc + 1) * (H // 4)] = jnp.zeros(
                (BT, H // 4), jnp.bfloat16)

    @pl.when(live)
    def _ffn():
        xs = xs_in_ref[...]
        wg = wg_ref[0].astype(jnp.bfloat16)
        wu = wu_ref[0].astype(jnp.bfloat16)
        g = jnp.dot(xs, wg, preferred_element_type=jnp.float32)
        u = jnp.dot(xs, wu, preferred_element_type=jnp.float32)
        h = g * jax.nn.sigmoid(g) * u
        yi = jnp.dot(h.astype(jnp.bfloat16), wd_ref[0].astype(jnp.bfloat16),
                     preferred_element_type=jnp.float32)

        @pl.when(i == 0)
        def _y_init():
            y_ref[...] = yi

        @pl.when(i > 0)
        def _y_acc():
            y_ref[...] += yi

        @pl.when(i == NI - 1)
        def _weight_rows():
            ws = ws_in_ref[...]
            for hc in range(2):
                sl = slice(hc * (H // 2), (hc + 1) * (H // 2))
                yw_ref[:, sl] = (y_ref[:, sl] * ws).astype(jnp.bfloat16)


def _scatter_kernel(sp_ref, posw_ref, yw_ref, out_ref):
    b = pl.program_id(0)
    live = sp_ref[1, b] == 1

    @pl.when(b == 0)
    def _zero_out():
        for zc in range(8):
            out_ref[:, zc * (H // 8):(zc + 1) * (H // 8)] = jnp.zeros(
                (T, H // 8), jnp.float32)

    @pl.when(live)
    def _scatter():
        posw = posw_ref[...]
        pos0 = posw[:, 0:1]
        pos1 = posw[:, 1:2]
        for c in range(BT // BC):
            jj = (jax.lax.broadcasted_iota(jnp.int32, (T, BC), 1)
                  + (b * BT + c * BC)).astype(jnp.float32)
            pmat = (jnp.where(pos0 == jj, 1.0, 0.0)
                    + jnp.where(pos1 == jj, 1.0, 0.0)).astype(jnp.bfloat16)
            for hc in range(2):
                sl = slice(hc * (H // 2), (hc + 1) * (H // 2))
                out_ref[:, sl] += jnp.dot(
                    pmat, yw_ref[c * BC:(c + 1) * BC, sl],
                    preferred_element_type=jnp.float32)


def kernel(hidden_states, gate_w, w_gate, w_up, w_down):
    B, S, Hd = hidden_states.shape
    x = hidden_states.reshape(-1, Hd)
    gw_pad = jnp.zeros((Hd, EP), dtype=gate_w.dtype).at[:, :E].set(gate_w)

    posw, counts, aux = pl.pallas_call(
        _router_kernel,
        out_shape=[
            jax.ShapeDtypeStruct((T, EP), jnp.float32),
            jax.ShapeDtypeStruct((1, EP), jnp.float32),
            jax.ShapeDtypeStruct((1, 1), jnp.float32),
        ],
        in_specs=[
            pl.BlockSpec((T, Hd), lambda: (0, 0)),
            pl.BlockSpec((Hd, EP), lambda: (0, 0)),
        ],
        out_specs=[
            pl.BlockSpec((T, EP), lambda: (0, 0)),
            pl.BlockSpec((1, EP), lambda: (0, 0)),
            pl.BlockSpec((1, 1), lambda: (0, 0)),
        ],
    )(x, gw_pad)

    pad_cnt = (jnp.ceil(counts[0, :E] / BT) * BT).astype(jnp.int32)
    ends = jnp.cumsum(pad_cnt)
    starts = jnp.arange(NB, dtype=jnp.int32) * BT
    be = jnp.sum((starts[:, None] >= ends[None, :]).astype(jnp.int32), axis=1)
    be = jnp.minimum(be, E - 1).astype(jnp.int32)
    live = (starts < ends[E - 1]).astype(jnp.int32)
    sp = jnp.stack([be, live])

    x_b = x.astype(jnp.bfloat16)
    gather_spec = pltpu.PrefetchScalarGridSpec(
        num_scalar_prefetch=1,
        grid=(NB,),
        in_specs=[
            pl.BlockSpec((T, EP), lambda b, sp: (0, 0)),
            pl.BlockSpec((T, Hd), lambda b, sp: (0, 0)),
        ],
        out_specs=[
            pl.BlockSpec((BT, Hd), lambda b, sp: (b, 0)),
            pl.BlockSpec((BT, 1), lambda b, sp: (b, 0)),
        ],
    )
    xs_all, ws_all = pl.pallas_call(
        _gather_kernel,
        grid_spec=gather_spec,
        out_shape=[
            jax.ShapeDtypeStruct((NP, Hd), jnp.bfloat16),
            jax.ShapeDtypeStruct((NP, 1), jnp.float32),
        ],
    )(sp, posw, x_b)

    grid_spec = pltpu.PrefetchScalarGridSpec(
        num_scalar_prefetch=1,
        grid=(NB, NI),
        in_specs=[
            pl.BlockSpec((BT, Hd), lambda b, i, sp: (b, 0)),
            pl.BlockSpec((BT, 1), lambda b, i, sp: (b, 0)),
            pl.BlockSpec((1, Hd, BI), lambda b, i, sp: (sp[0, b], 0, i * sp[1, b])),
            pl.BlockSpec((1, Hd, BI), lambda b, i, sp: (sp[0, b], 0, i * sp[1, b])),
            pl.BlockSpec((1, BI, Hd), lambda b, i, sp: (sp[0, b], i * sp[1, b], 0)),
        ],
        out_specs=pl.BlockSpec((BT, Hd), lambda b, i, sp: (b, 0)),
        scratch_shapes=[
            pltpu.VMEM((BT, Hd), jnp.float32),
        ],
    )
    yw = pl.pallas_call(
        _ffn_kernel,
        grid_spec=grid_spec,
        out_shape=jax.ShapeDtypeStruct((NP, Hd), jnp.bfloat16),
    )(sp, xs_all, ws_all, w_gate, w_up, w_down)

    scatter_spec = pltpu.PrefetchScalarGridSpec(
        num_scalar_prefetch=1,
        grid=(NB,),
        in_specs=[
            pl.BlockSpec((T, EP), lambda b, sp: (0, 0)),
            pl.BlockSpec((BT, Hd), lambda b, sp: (b, 0)),
        ],
        out_specs=pl.BlockSpec((T, Hd), lambda b, sp: (0, 0)),
    )
    out = pl.pallas_call(
        _scatter_kernel,
        grid_spec=scatter_spec,
        out_shape=jax.ShapeDtypeStruct((T, Hd), jnp.float32),
    )(sp, posw, yw)

    return out.reshape(B, S, Hd), aux.reshape(())
